# Initial kernel scaffold; baseline (speedup 1.0000x reference)
import jax
import jax.numpy as jnp
from jax.experimental import pallas as pl


def kernel(keys, queue, ptr, filled):
    raise NotImplementedError("write your pallas kernel here")


# speedup vs baseline: 3.4422x; 3.4422x over previous
import jax
import jax.numpy as jnp
from jax.experimental import pallas as pl
from jax.experimental.pallas import tpu as pltpu

_DIM = 128
_COLS = 4096


def _enqueue_body(ptr_ref, keys_ref, queue_ref, out_ref):
    j = pl.program_id(0)
    slot_blk = ptr_ref[0] // _COLS

    @pl.when(j != slot_blk)
    def _copy():
        out_ref[...] = queue_ref[...]

    @pl.when(j == slot_blk)
    def _enqueue():
        k = keys_ref[...]
        norm = jnp.sqrt(jnp.sum(k * k, axis=1, keepdims=True))
        kn = k / jnp.maximum(norm, 1e-12)
        out_ref[...] = kn.T


def kernel(keys, queue, ptr, filled):
    keys = keys.astype(jnp.float32)
    b, dim = keys.shape
    dim2, kq = queue.shape
    assert dim == _DIM and dim2 == _DIM and b == _COLS and kq % _COLS == 0
    nblk = kq // _COLS

    ptr_arr = jnp.asarray(ptr, jnp.int32).reshape(1)

    grid_spec = pltpu.PrefetchScalarGridSpec(
        num_scalar_prefetch=1,
        grid=(nblk,),
        in_specs=[
            pl.BlockSpec((b, dim), lambda j, p: (0, 0)),
            pl.BlockSpec((dim, _COLS), lambda j, p: (0, j)),
        ],
        out_specs=pl.BlockSpec((dim, _COLS), lambda j, p: (0, j)),
    )

    new_queue = pl.pallas_call(
        _enqueue_body,
        grid_spec=grid_spec,
        out_shape=jax.ShapeDtypeStruct((dim, kq), jnp.float32),
    )(ptr_arr, keys, queue)

    new_ptr = jnp.reshape((ptr + b) % kq, (1,)).astype(jnp.int32)
    new_filled = jnp.reshape(jnp.minimum(filled + b, kq), (1,)).astype(jnp.int32)
    return new_queue, new_ptr, new_filled
